# baseline (speedup 1.0000x reference)
import functools

import jax
import jax.numpy as jnp
from jax import lax
from jax.experimental import pallas as pl
from jax.experimental.pallas import tpu as pltpu
from jax.experimental.pallas import tpu_sc as plsc


def kernel(x, seq_emb_weight, pos_encoding):
    B, S = x.shape
    V, D = seq_emb_weight.shape
    N = B * S

    info = plsc.get_sparse_core_info()
    NC, NS, L = info.num_cores, info.num_subcores, info.num_lanes
    NW = NC * NS

    NBUF = 4
    GD = 2
    seq_per_w = B // NW
    n_groups = seq_per_w // NBUF
    G1 = 128
    G2 = S - G1

    pos = pos_encoding[:S]

    mesh = plsc.VectorSubcoreMesh(core_axis_name="c", subcore_axis_name="s",
                                  num_cores=NC)

    @functools.partial(
        pl.kernel,
        mesh=mesh,
        out_type=jax.ShapeDtypeStruct((N, D), jnp.float32),
        compiler_params=pltpu.CompilerParams(use_tc_tiling_on_sc=False),
        scratch_types=[
            pltpu.VMEM((NBUF, 2, G1), jnp.int32),
            pltpu.VMEM((NBUF, S, D), jnp.float32),
            pltpu.VMEM((S, D), jnp.float32),
            pltpu.SemaphoreType.DMA((NBUF,)),
            pltpu.SemaphoreType.DMA((NBUF,)),
            pltpu.SemaphoreType.DMA((NBUF,)),
        ],
    )
    def emb_pos_kernel(table_hbm, idx_hbm, pos_hbm, out_hbm,
                       idx_v, rows_v, pos_v, sem_idx, sem_g, sem_out):
        wid = lax.axis_index("s") * NC + lax.axis_index("c")
        seq0 = wid * seq_per_w

        pltpu.sync_copy(pos_hbm, pos_v)

        def idx_copies(s, b):
            fb = (seq0 + s) * S
            return (
                pltpu.make_async_copy(
                    idx_hbm.at[pl.ds(fb, G1)], idx_v.at[b, 0], sem_idx.at[b]),
                pltpu.make_async_copy(
                    idx_hbm.at[pl.ds(fb + G1, G2)],
                    idx_v.at[b, 1, pl.ds(0, G2)], sem_idx.at[b]),
            )

        def gather_copies(b):
            return (
                pltpu.make_async_copy(
                    table_hbm.at[idx_v.at[b, 0]],
                    rows_v.at[b, pl.ds(0, G1)], sem_g.at[b]),
                pltpu.make_async_copy(
                    table_hbm.at[idx_v.at[b, 1, pl.ds(0, G2)]],
                    rows_v.at[b, pl.ds(G1, G2)], sem_g.at[b]),
            )

        def out_copy(s, b):
            return pltpu.make_async_copy(
                rows_v.at[b], out_hbm.at[pl.ds((seq0 + s) * S, S)],
                sem_out.at[b])

        def start2(copies):
            copies[0].start()
            copies[1].start()

        def wait2(copies):
            copies[0].wait()
            copies[1].wait()

        for b in range(NBUF):
            start2(idx_copies(b, b))
        for s in range(GD):
            wait2(idx_copies(s, s))
            start2(gather_copies(s))

        def group_body(g, carry):
            for b in range(NBUF):
                s = g * NBUF + b

                b3 = (b + GD) % NBUF

                @pl.when(s + GD < seq_per_w)
                def _():
                    wait2(idx_copies(s + GD, b3))

                    @pl.when(s + GD >= NBUF)
                    def _():
                        out_copy(0, b3).wait()

                    start2(gather_copies(b3))

                wait2(gather_copies(b))

                @pl.when(s + NBUF < seq_per_w)
                def _():
                    start2(idx_copies(s + NBUF, b))

                out_copy(s, b).start()
            return carry

        lax.fori_loop(0, n_groups, group_body, 0)

        for b in range(NBUF):
            out_copy(0, b).wait()

    flat = emb_pos_kernel(seq_emb_weight, x.reshape(N), pos)
    return flat.reshape(B, S, D) + pos[None, :, :]


# speedup vs baseline: 1.1023x; 1.1023x over previous
import functools

import jax
import jax.numpy as jnp
from jax import lax
from jax.experimental import pallas as pl
from jax.experimental.pallas import tpu as pltpu
from jax.experimental.pallas import tpu_sc as plsc


def kernel(x, seq_emb_weight, pos_encoding):
    B, S = x.shape
    V, D = seq_emb_weight.shape
    N = B * S

    info = plsc.get_sparse_core_info()
    NC, NS, L = info.num_cores, info.num_subcores, info.num_lanes
    NW = NC * NS

    NBUF = 4
    GD = 2
    seq_per_w = B // NW
    n_groups = seq_per_w // NBUF
    G1 = 128
    G2 = S - G1

    pos = pos_encoding[:S]

    mesh = plsc.VectorSubcoreMesh(core_axis_name="c", subcore_axis_name="s",
                                  num_cores=NC)

    @functools.partial(
        pl.kernel,
        mesh=mesh,
        out_type=jax.ShapeDtypeStruct((B, S, D), jnp.float32),
        compiler_params=pltpu.CompilerParams(use_tc_tiling_on_sc=False),
        scratch_types=[
            pltpu.VMEM((NBUF, 2, G1), jnp.int32),
            pltpu.VMEM((NBUF, S, D), jnp.float32),
            pltpu.VMEM((S, D), jnp.float32),
            pltpu.SemaphoreType.DMA((NBUF,)),
            pltpu.SemaphoreType.DMA((NBUF,)),
            pltpu.SemaphoreType.DMA((NBUF,)),
        ],
    )
    def emb_pos_kernel(table_hbm, idx_hbm, pos_hbm, out_hbm,
                       idx_v, rows_v, pos_v, sem_idx, sem_g, sem_out):
        wid = lax.axis_index("s") * NC + lax.axis_index("c")
        seq0 = wid * seq_per_w

        pltpu.sync_copy(pos_hbm, pos_v)

        def idx_copies(s, b):
            fb = (seq0 + s) * S
            return (
                pltpu.make_async_copy(
                    idx_hbm.at[pl.ds(fb, G1)], idx_v.at[b, 0], sem_idx.at[b]),
                pltpu.make_async_copy(
                    idx_hbm.at[pl.ds(fb + G1, G2)],
                    idx_v.at[b, 1, pl.ds(0, G2)], sem_idx.at[b]),
            )

        def gather_copies(b):
            return (
                pltpu.make_async_copy(
                    table_hbm.at[idx_v.at[b, 0]],
                    rows_v.at[b, pl.ds(0, G1)], sem_g.at[b]),
                pltpu.make_async_copy(
                    table_hbm.at[idx_v.at[b, 1, pl.ds(0, G2)]],
                    rows_v.at[b, pl.ds(G1, G2)], sem_g.at[b]),
            )

        def out_copy(s, b):
            return pltpu.make_async_copy(
                rows_v.at[b], out_hbm.at[seq0 + s], sem_out.at[b])

        def start2(copies):
            copies[0].start()
            copies[1].start()

        def wait2(copies):
            copies[0].wait()
            copies[1].wait()

        for b in range(NBUF):
            start2(idx_copies(b, b))
        for s in range(GD):
            wait2(idx_copies(s, s))
            start2(gather_copies(s))

        def group_body(g, carry):
            for b in range(NBUF):
                s = g * NBUF + b

                b3 = (b + GD) % NBUF

                @pl.when(s + GD < seq_per_w)
                def _():
                    wait2(idx_copies(s + GD, b3))

                    @pl.when(s + GD >= NBUF)
                    def _():
                        out_copy(0, b3).wait()

                    start2(gather_copies(b3))

                wait2(gather_copies(b))

                @pl.when(s + NBUF < seq_per_w)
                def _():
                    start2(idx_copies(s + NBUF, b))

                def row_body(i, rcarry):
                    for rr in range(4):
                        r = i * 4 + rr
                        for j in range(D // L):
                            plsc.addupdate(rows_v.at[b, r, pl.ds(j * L, L)],
                                           pos_v[r, pl.ds(j * L, L)])
                    return rcarry

                lax.fori_loop(0, S // 4, row_body, 0)

                out_copy(s, b).start()
            return carry

        lax.fori_loop(0, n_groups, group_body, 0)

        for b in range(NBUF):
            out_copy(0, b).wait()

    return emb_pos_kernel(seq_emb_weight, x.reshape(N), pos)
